# baseline (speedup 1.0000x reference)
import functools

import jax
import jax.numpy as jnp
from jax import lax
from jax.experimental import pallas as pl
from jax.experimental.pallas import tpu as pltpu
from jax.experimental.pallas import tpu_sc as plsc

N_NODES = 10000
N_EDGES = 320000
D_FEAT = 128

NT, L = 16, 16
NPAD = 10240
WIN = 80
RPT = N_EDGES // (NT * WIN)
EPT = N_EDGES // NT
NSL = NPAD // NT

_mesh = plsc.VectorSubcoreMesh(
    core_axis_name="c", subcore_axis_name="s", num_cores=1, num_subcores=NT
)
_sc_params = pltpu.CompilerParams(needs_layout_passes=False)


def _rsqrt16(d):
    i = lax.bitcast_convert_type(d, jnp.int32)
    y = lax.bitcast_convert_type(jnp.int32(0x5F3759DF) - (i >> 1), jnp.float32)
    for _ in range(3):
        y = y * (1.5 - 0.5 * d * y * y)
    return y


def _mish16(z):
    t = 1.0 + jnp.exp(z)
    tt = t * t
    return jnp.where(z > 15.0, z, z * (tt - 1.0) / (tt + 1.0))


@functools.partial(
    pl.kernel,
    out_type=jax.ShapeDtypeStruct((NPAD,), jnp.float32),
    mesh=_mesh,
    scratch_types=[
        pltpu.VMEM((RPT, WIN), jnp.int32),
        pltpu.VMEM((EPT,), jnp.int32),
        pltpu.VMEM((RPT, WIN), jnp.float32),
        pltpu.VMEM((NPAD,), jnp.float32),
        pltpu.VMEM((NSL,), jnp.float32),
        pltpu.VMEM((NSL,), jnp.float32),
        pltpu.VMEM((NSL,), jnp.float32),
        pltpu.VMEM((NSL,), jnp.float32),
        pltpu.VMEM((L,), jnp.float32),
        pltpu.VMEM((NSL,), jnp.float32),
        pltpu.VMEM_SHARED((NPAD,), jnp.float32),
        pltpu.VMEM_SHARED((NPAD,), jnp.float32),
        pltpu.SemaphoreType.DMA,
    ],
    compiler_params=_sc_params,
)
def _sc_gcn(col_hbm, row_hbm, attr_hbm, h_hbm, b_hbm, out_hbm,
            colv, rowv, attrv, gv, hv, dv, gsl, sv, bv, zv,
            acc_sh, g_sh, sem):
    sid = lax.axis_index("s")
    nbase = sid * NSL

    c1 = pltpu.async_copy(col_hbm.at[sid], colv, sem)
    c2 = pltpu.async_copy(attr_hbm.at[sid], attrv, sem)
    c3 = pltpu.async_copy(row_hbm.at[pl.ds(sid * EPT, EPT)], rowv, sem)
    c4 = pltpu.async_copy(h_hbm.at[pl.ds(nbase, NSL)], hv, sem)
    c5 = pltpu.async_copy(b_hbm, bv, sem)

    @pl.loop(0, NSL // L)
    def _(i):
        zv[pl.ds(i * L, L)] = jnp.zeros((L,), jnp.float32)

    pltpu.sync_copy(zv, acc_sh.at[pl.ds(nbase, NSL)])
    c1.wait()
    c2.wait()
    plsc.subcore_barrier()

    @pl.loop(0, RPT)
    def _(j):
        pltpu.async_copy(attrv.at[j], acc_sh.at[colv.at[j]], sem, add=True)

    @pl.loop(0, RPT)
    def _(j):
        pltpu.make_async_copy(attrv.at[j], acc_sh.at[colv.at[j]], sem).wait()

    plsc.subcore_barrier()

    c4.wait()
    pltpu.sync_copy(acc_sh.at[pl.ds(nbase, NSL)], dv)
    pltpu.sync_copy(zv, acc_sh.at[pl.ds(nbase, NSL)])

    @pl.loop(0, NSL // L)
    def _(i):
        sl = pl.ds(i * L, L)
        y = _rsqrt16(dv[sl] + 1.0)
        dv[sl] = y
        gsl[sl] = hv[sl] * y

    pltpu.sync_copy(gsl, g_sh.at[pl.ds(nbase, NSL)])
    c3.wait()
    plsc.subcore_barrier()

    pltpu.sync_copy(g_sh, gv)

    @pl.loop(0, RPT)
    def _(j):
        @pl.loop(0, WIN // L)
        def _(k):
            sl = pl.ds(k * L, L)
            idx = rowv[pl.ds(j * WIN + k * L, L)]
            attrv[j, sl] = plsc.load_gather(gv, [idx]) * attrv[j, sl]

        pltpu.async_copy(attrv.at[j], acc_sh.at[colv.at[j]], sem, add=True)

    @pl.loop(0, RPT)
    def _(j):
        pltpu.make_async_copy(attrv.at[j], acc_sh.at[colv.at[j]], sem).wait()

    plsc.subcore_barrier()

    c5.wait()
    pltpu.sync_copy(acc_sh.at[pl.ds(nbase, NSL)], sv)

    @pl.loop(0, NSL // L)
    def _(i):
        sl = pl.ds(i * L, L)
        z = bv[...] + dv[sl] * (sv[sl] + gsl[sl])
        sv[sl] = _mish16(z)

    pltpu.sync_copy(sv, out_hbm.at[pl.ds(nbase, NSL)])


def _mv_body(w_ref, x_ref, o_ref):
    o_ref[...] = lax.dot_general(
        w_ref[...], x_ref[...], (((1,), (1,)), ((), ())),
        preferred_element_type=jnp.float32,
    )


def kernel(x, edge_index, attrs, W, b):
    row = edge_index[0].astype(jnp.int32)
    col = edge_index[1].astype(jnp.int32)
    col3d = col.reshape(NT, RPT, WIN)
    attr3d = attrs.reshape(NT, RPT, WIN)

    h = pl.pallas_call(
        _mv_body, out_shape=jax.ShapeDtypeStruct((1, N_NODES), jnp.float32)
    )(W, x)
    h_pad = jnp.pad(h.reshape(-1), (0, NPAD - N_NODES))
    b16 = jnp.broadcast_to(b.astype(jnp.float32), (L,))

    out = _sc_gcn(col3d, row, attr3d, h_pad, b16)
    return out[:N_NODES].reshape(1, N_NODES)
